# Initial kernel scaffold; baseline (speedup 1.0000x reference)
import jax
import jax.numpy as jnp
from jax.experimental import pallas as pl


def kernel(x, edge_index, batch_ids, aoa, mach_or_reynolds, marker_inds, nodes, Wp0, bp0, Wp1, bp1, Wp2, bp2, We0, be0, We1, be1, We2, be2):
    raise NotImplementedError("write your pallas kernel here")


# speedup vs baseline: 6.0775x; 6.0775x over previous
import functools

import jax
import jax.numpy as jnp
from jax import lax
from jax.experimental import pallas as pl
from jax.experimental.pallas import tpu as pltpu, tpu_sc as plsc

N_FINE = 50000
NP = 50048
E = 800000
EP = 819200
CH = 128
CHUNKS = EP // (32 * CH)
ROWS_PER_TILE = NP // 16
H = 64
HH = 32
DUMMY_COL = N_FINE + 8


def _sc_agg(table, row2, col2, zeros):
    mesh = plsc.VectorSubcoreMesh(core_axis_name="c", subcore_axis_name="s")

    @functools.partial(
        pl.kernel,
        mesh=mesh,
        out_type=jax.ShapeDtypeStruct((2, NP, HH), jnp.float32),
        scratch_types=[
            pltpu.VMEM((CH,), jnp.int32),
            pltpu.VMEM((CH,), jnp.int32),
            pltpu.VMEM((CH, HH), jnp.float32),
            pltpu.VMEM_SHARED((NP, HH), jnp.float32),
            pltpu.SemaphoreType.DMA,
        ],
        compiler_params=pltpu.CompilerParams(use_tc_tiling_on_sc=False),
    )
    def k(tab_hbm, row_hbm, col_hbm, z_hbm, out_hbm, row_v, col_v, rows_v, acc, sem):
        c = lax.axis_index("c")
        s = lax.axis_index("s")
        wid = c * 16 + s
        pltpu.sync_copy(z_hbm.at[pl.ds(s * ROWS_PER_TILE, ROWS_PER_TILE)],
                        acc.at[pl.ds(s * ROWS_PER_TILE, ROWS_PER_TILE)])
        plsc.subcore_barrier()
        base0 = wid * CHUNKS * CH

        def body(j, carry):
            base = base0 + j * CH
            pltpu.sync_copy(row_hbm.at[pl.ds(base, CH)], row_v)
            pltpu.sync_copy(col_hbm.at[pl.ds(base, CH)], col_v)
            pltpu.async_copy(tab_hbm.at[row_v], rows_v, sem).wait()
            pltpu.sync_copy(rows_v, acc.at[col_v], add=True)
            return carry

        lax.fori_loop(0, CHUNKS, body, 0)
        plsc.subcore_barrier()
        pltpu.sync_copy(acc.at[pl.ds(s * ROWS_PER_TILE, ROWS_PER_TILE)],
                        out_hbm.at[c, pl.ds(s * ROWS_PER_TILE, ROWS_PER_TILE)])

    return k(table, row2, col2, zeros)


def _mm_scale(x, w, dinv, rb=1000):
    n, kd = x.shape
    ho = w.shape[1]

    def body(x_ref, w_ref, d_ref, o_ref):
        o_ref[...] = d_ref[...] * jnp.dot(
            x_ref[...], w_ref[...], preferred_element_type=jnp.float32)

    return pl.pallas_call(
        body,
        grid=(n // rb,),
        in_specs=[
            pl.BlockSpec((rb, kd), lambda i: (i, 0)),
            pl.BlockSpec((kd, ho), lambda i: (0, 0)),
            pl.BlockSpec((rb, 1), lambda i: (i, 0)),
        ],
        out_specs=pl.BlockSpec((rb, ho), lambda i: (i, 0)),
        out_shape=jax.ShapeDtypeStruct((n, ho), jnp.float32),
    )(x, w, dinv)


def _combine(a0, a1, g, dinv, b, relu, rb=1000):
    n = g.shape[0]

    def body(a0_ref, a1_ref, g_ref, d_ref, b_ref, o_ref):
        lo = a0_ref[0] + a0_ref[1]
        hi = a1_ref[0] + a1_ref[1]
        agg = jnp.concatenate([lo, hi], axis=1)
        y = d_ref[...] * (agg + g_ref[...]) + b_ref[...]
        if relu:
            y = jnp.maximum(y, 0.0)
        o_ref[...] = y

    return pl.pallas_call(
        body,
        grid=(n // rb,),
        in_specs=[
            pl.BlockSpec((2, rb, HH), lambda i: (0, i, 0)),
            pl.BlockSpec((2, rb, HH), lambda i: (0, i, 0)),
            pl.BlockSpec((rb, H), lambda i: (i, 0)),
            pl.BlockSpec((rb, 1), lambda i: (i, 0)),
            pl.BlockSpec((1, H), lambda i: (0, 0)),
        ],
        out_specs=pl.BlockSpec((rb, H), lambda i: (i, 0)),
        out_shape=jax.ShapeDtypeStruct((n, H), jnp.float32),
    )(a0, a1, g, dinv, b)


def _combine1(a0, g, dinv, b, rb=1000):
    n = g.shape[0]

    def body(a0_ref, g_ref, d_ref, b_ref, o_ref):
        agg = a0_ref[0] + a0_ref[1]
        o_ref[...] = d_ref[...] * (agg + g_ref[...]) + b_ref[...]

    return pl.pallas_call(
        body,
        grid=(n // rb,),
        in_specs=[
            pl.BlockSpec((2, rb, HH), lambda i: (0, i, 0)),
            pl.BlockSpec((rb, HH), lambda i: (i, 0)),
            pl.BlockSpec((rb, 1), lambda i: (i, 0)),
            pl.BlockSpec((1, HH), lambda i: (0, 0)),
        ],
        out_specs=pl.BlockSpec((rb, HH), lambda i: (i, 0)),
        out_shape=jax.ShapeDtypeStruct((n, HH), jnp.float32),
    )(a0, g, dinv, b)


def _dinv_from_deg(d, rb=1000):
    n = N_FINE

    def body(d_ref, o_ref):
        deg = d_ref[0, :, 0:1] + d_ref[1, :, 0:1] + 1.0
        o_ref[...] = lax.rsqrt(deg)

    return pl.pallas_call(
        body,
        grid=(n // rb,),
        in_specs=[pl.BlockSpec((2, rb, HH), lambda i: (0, i, 0))],
        out_specs=pl.BlockSpec((rb, 1), lambda i: (i, 0)),
        out_shape=jax.ShapeDtypeStruct((n, 1), jnp.float32),
    )(d)


def _sdf(fine0, mkt, rb=512):
    n = fine0.shape[0]
    nm = mkt.shape[1]

    def body(f_ref, m_ref, o_ref):
        fx = f_ref[:, 0:1]
        fy = f_ref[:, 1:2]
        mx = m_ref[0:1, :]
        my = m_ref[1:2, :]
        d2 = (fx - mx) ** 2 + (fy - my) ** 2
        o_ref[...] = jnp.min(jnp.sqrt(d2 + 1e-12), axis=1, keepdims=True)

    return pl.pallas_call(
        body,
        grid=(n // rb,),
        in_specs=[
            pl.BlockSpec((rb, 2), lambda i: (i, 0)),
            pl.BlockSpec((2, nm), lambda i: (0, 0)),
        ],
        out_specs=pl.BlockSpec((rb, 1), lambda i: (i, 0)),
        out_shape=jax.ShapeDtypeStruct((n, 1), jnp.float32),
    )(fine0, mkt)


def _knn_interp(fn, nodest, cyflat, n_per, nb, rb=400):
    n = fn.shape[0]
    nc = nodest.shape[1]

    def body(f_ref, nd_ref, cy_ref, o_ref):
        i = pl.program_id(0)
        fx = f_ref[:, 0:1]
        fy = f_ref[:, 1:2]
        nx = nd_ref[0:1, :]
        ny = nd_ref[1:2, :]
        d2 = (fx - nx) ** 2 + (fy - ny) ** 2
        iot = lax.broadcasted_iota(jnp.int32, (rb, nc), 1)
        wacc = jnp.zeros((rb, nc), jnp.float32)
        wsum = jnp.zeros((rb, 1), jnp.float32)
        d2w = d2
        for _ in range(3):
            m = jnp.min(d2w, axis=1, keepdims=True)
            idx = jnp.min(jnp.where(d2w == m, iot, nc), axis=1, keepdims=True)
            oh = iot == idx
            w = 1.0 / (m + 1e-16)
            wacc = wacc + jnp.where(oh, w, 0.0)
            wsum = wsum + w
            d2w = jnp.where(oh, 1e30, d2w)
        row_batch = (i * rb + lax.broadcasted_iota(jnp.int32, (rb, 1), 0)) // n_per
        col_batch = lax.broadcasted_iota(jnp.int32, (rb, nb * nc), 1) // nc
        gate = col_batch == row_batch
        wacc4 = jnp.where(gate, jnp.tile(wacc, (1, nb)), 0.0)
        o_ref[...] = jnp.dot(wacc4, cy_ref[...],
                             preferred_element_type=jnp.float32) / wsum

    return pl.pallas_call(
        body,
        grid=(n // rb,),
        in_specs=[
            pl.BlockSpec((rb, 2), lambda i: (i, 0)),
            pl.BlockSpec((2, nc), lambda i: (0, 0)),
            pl.BlockSpec((nb * nc, 3), lambda i: (0, 0)),
        ],
        out_specs=pl.BlockSpec((rb, 3), lambda i: (i, 0)),
        out_shape=jax.ShapeDtypeStruct((n, 3), jnp.float32),
    )(fn, nodest, cyflat)


def kernel(x, edge_index, batch_ids, aoa, mach_or_reynolds, marker_inds, nodes,
           Wp0, bp0, Wp1, bp1, Wp2, bp2, We0, be0, We1, be1, We2, be2):
    batch_size = aoa.shape[0]
    n_per = x.shape[0] // batch_size

    row = edge_index[0].astype(jnp.int32)
    col = edge_index[1].astype(jnp.int32)
    pad = EP - E
    row2 = jnp.concatenate([row, jnp.zeros((pad,), jnp.int32)])
    col2 = jnp.concatenate([col, jnp.full((pad,), DUMMY_COL, jnp.int32)])
    zeros = jnp.zeros((NP, HH), jnp.float32)

    ones_tab = jnp.ones((N_FINE, HH), jnp.float32)
    deg_parts = _sc_agg(ones_tab, row2, col2, zeros)[:, :N_FINE, :]
    dinv = _dinv_from_deg(deg_parts)

    def conv(xin, w, b, relu):
        g = _mm_scale(xin, w, dinv)
        a0 = _sc_agg(g[:, :HH], row2, col2, zeros)[:, :N_FINE, :]
        a1 = _sc_agg(g[:, HH:], row2, col2, zeros)[:, :N_FINE, :]
        return _combine(a0, a1, g, dinv, b.reshape(1, H), relu)

    mkt = x[marker_inds, :2].T
    n_pad = ((n_per + 511) // 512) * 512
    fine0p = jnp.pad(x[:n_per, :2], ((0, n_pad - n_per), (0, 0)))
    sdf = _sdf(fine0p, mkt)[:n_per]
    fine_x = jnp.concatenate([x, jnp.tile(sdf, (batch_size, 1))], axis=1)

    fine_x = conv(fine_x, Wp0, bp0, True)
    fine_x = conv(fine_x, Wp1, bp1, True)
    fine_x = conv(fine_x, Wp2, bp2, True)

    cx = nodes[:, 0]
    cy = nodes[:, 1]
    a = aoa[:, None]
    m = mach_or_reynolds[:, None]
    y0 = jnp.sin(3.0 * cx)[None, :] * a + m * cy[None, :]
    y1 = jnp.cos(3.0 * cy)[None, :] * m + a * cx[None, :]
    y2 = jnp.sin(cx + cy)[None, :] * (a + m)
    cyflat = jnp.stack([y0.reshape(-1), y1.reshape(-1), y2.reshape(-1)], axis=1)

    fine_y = _knn_interp(x[:, :2], nodes.T, cyflat, n_per, batch_size)
    fine_y = jnp.concatenate([fine_y, fine_x], axis=1)

    fine_y = conv(fine_y, We0, be0, True)
    fine_y = conv(fine_y, We1, be1, True)

    out_w = We2.shape[1]
    w32 = jnp.pad(We2, ((0, 0), (0, HH - out_w)))
    b32 = jnp.pad(be2, (0, HH - out_w))
    g = _mm_scale(fine_y, w32, dinv)
    a0 = _sc_agg(g, row2, col2, zeros)[:, :N_FINE, :]
    out = _combine1(a0, g, dinv, b32.reshape(1, HH))
    return out[:, :out_w]


# speedup vs baseline: 9.2998x; 1.5302x over previous
import functools

import jax
import jax.numpy as jnp
from jax import lax
from jax.experimental import pallas as pl
from jax.experimental.pallas import tpu as pltpu, tpu_sc as plsc

N_FINE = 50000
NP = 50048
E = 800000
EP = 819200
CH = 128
CHUNKS = EP // (32 * CH)
GRP = 8
NGRP = CHUNKS // GRP
ROWS_PER_TILE = NP // 16
H = 64
HH = 32
DUMMY_COL = N_FINE + 8


def _sc_agg(table, row2, col2, zeros):
    mesh = plsc.VectorSubcoreMesh(core_axis_name="c", subcore_axis_name="s")

    @functools.partial(
        pl.kernel,
        mesh=mesh,
        out_type=jax.ShapeDtypeStruct((2, NP, HH), jnp.float32),
        scratch_types=[
            pltpu.VMEM((2, GRP, CH), jnp.int32),
            pltpu.VMEM((2, GRP, CH), jnp.int32),
            pltpu.VMEM((2, CH, HH), jnp.float32),
            pltpu.VMEM_SHARED((NP, HH), jnp.float32),
            pltpu.SemaphoreType.DMA,
            pltpu.SemaphoreType.DMA,
        ],
        compiler_params=pltpu.CompilerParams(use_tc_tiling_on_sc=False),
    )
    def k(tab_hbm, row_hbm, col_hbm, z_hbm, out_hbm,
          row_g, col_g, rows_v, acc, isem, gsem):
        c = lax.axis_index("c")
        s = lax.axis_index("s")
        wid = c * 16 + s
        pltpu.sync_copy(z_hbm.at[pl.ds(s * ROWS_PER_TILE, ROWS_PER_TILE)],
                        acc.at[pl.ds(s * ROWS_PER_TILE, ROWS_PER_TILE)])
        plsc.subcore_barrier()
        grow0 = wid * CHUNKS

        def load_idx(g, slot):
            off = grow0 + g * GRP
            pltpu.async_copy(row_hbm.at[pl.ds(off, GRP)], row_g.at[slot], isem)
            pltpu.async_copy(col_hbm.at[pl.ds(off, GRP)], col_g.at[slot], isem)

        load_idx(0, 0)

        def group(g, carry):
            slot = lax.rem(g, 2)
            pltpu.make_async_copy(row_hbm.at[pl.ds(0, GRP)], row_g.at[slot], isem).wait()
            pltpu.make_async_copy(col_hbm.at[pl.ds(0, GRP)], col_g.at[slot], isem).wait()

            @pl.when(g < NGRP - 1)
            def _():
                load_idx(g + 1, 1 - slot)

            pltpu.async_copy(tab_hbm.at[row_g.at[slot, 0]], rows_v.at[0], gsem)
            for b in range(GRP):
                if b + 1 < GRP:
                    pltpu.async_copy(tab_hbm.at[row_g.at[slot, b + 1]],
                                     rows_v.at[(b + 1) % 2], gsem)
                pltpu.make_async_copy(tab_hbm.at[row_g.at[slot, b]],
                                      rows_v.at[b % 2], gsem).wait()
                pltpu.sync_copy(rows_v.at[b % 2], acc.at[col_g.at[slot, b]], add=True)
            return carry

        lax.fori_loop(0, NGRP, group, 0)
        plsc.subcore_barrier()
        pltpu.sync_copy(acc.at[pl.ds(s * ROWS_PER_TILE, ROWS_PER_TILE)],
                        out_hbm.at[c, pl.ds(s * ROWS_PER_TILE, ROWS_PER_TILE)])

    return k(table, row2, col2, zeros)


def _mm_scale(x, w, dinv, rb=1000):
    n, kd = x.shape
    ho = w.shape[1]

    def body(x_ref, w_ref, d_ref, o_ref):
        o_ref[...] = d_ref[...] * jnp.dot(
            x_ref[...], w_ref[...], preferred_element_type=jnp.float32)

    return pl.pallas_call(
        body,
        grid=(n // rb,),
        in_specs=[
            pl.BlockSpec((rb, kd), lambda i: (i, 0)),
            pl.BlockSpec((kd, ho), lambda i: (0, 0)),
            pl.BlockSpec((rb, 1), lambda i: (i, 0)),
        ],
        out_specs=pl.BlockSpec((rb, ho), lambda i: (i, 0)),
        out_shape=jax.ShapeDtypeStruct((n, ho), jnp.float32),
    )(x, w, dinv)


def _combine(a0, a1, g, dinv, b, relu, rb=1000):
    n = g.shape[0]

    def body(a0_ref, a1_ref, g_ref, d_ref, b_ref, o_ref):
        lo = a0_ref[0] + a0_ref[1]
        hi = a1_ref[0] + a1_ref[1]
        agg = jnp.concatenate([lo, hi], axis=1)
        y = d_ref[...] * (agg + g_ref[...]) + b_ref[...]
        if relu:
            y = jnp.maximum(y, 0.0)
        o_ref[...] = y

    return pl.pallas_call(
        body,
        grid=(n // rb,),
        in_specs=[
            pl.BlockSpec((2, rb, HH), lambda i: (0, i, 0)),
            pl.BlockSpec((2, rb, HH), lambda i: (0, i, 0)),
            pl.BlockSpec((rb, H), lambda i: (i, 0)),
            pl.BlockSpec((rb, 1), lambda i: (i, 0)),
            pl.BlockSpec((1, H), lambda i: (0, 0)),
        ],
        out_specs=pl.BlockSpec((rb, H), lambda i: (i, 0)),
        out_shape=jax.ShapeDtypeStruct((n, H), jnp.float32),
    )(a0, a1, g, dinv, b)


def _combine1(a0, g, dinv, b, rb=1000):
    n = g.shape[0]

    def body(a0_ref, g_ref, d_ref, b_ref, o_ref):
        agg = a0_ref[0] + a0_ref[1]
        o_ref[...] = d_ref[...] * (agg + g_ref[...]) + b_ref[...]

    return pl.pallas_call(
        body,
        grid=(n // rb,),
        in_specs=[
            pl.BlockSpec((2, rb, HH), lambda i: (0, i, 0)),
            pl.BlockSpec((rb, HH), lambda i: (i, 0)),
            pl.BlockSpec((rb, 1), lambda i: (i, 0)),
            pl.BlockSpec((1, HH), lambda i: (0, 0)),
        ],
        out_specs=pl.BlockSpec((rb, HH), lambda i: (i, 0)),
        out_shape=jax.ShapeDtypeStruct((n, HH), jnp.float32),
    )(a0, g, dinv, b)


def _dinv_from_deg(d, rb=1000):
    n = N_FINE

    def body(d_ref, o_ref):
        deg = d_ref[0, :, 0:1] + d_ref[1, :, 0:1] + 1.0
        o_ref[...] = lax.rsqrt(deg)

    return pl.pallas_call(
        body,
        grid=(n // rb,),
        in_specs=[pl.BlockSpec((2, rb, HH), lambda i: (0, i, 0))],
        out_specs=pl.BlockSpec((rb, 1), lambda i: (i, 0)),
        out_shape=jax.ShapeDtypeStruct((n, 1), jnp.float32),
    )(d)


def _sdf(fine0, mkt, rb=512):
    n = fine0.shape[0]
    nm = mkt.shape[1]

    def body(f_ref, m_ref, o_ref):
        fx = f_ref[:, 0:1]
        fy = f_ref[:, 1:2]
        mx = m_ref[0:1, :]
        my = m_ref[1:2, :]
        d2 = (fx - mx) ** 2 + (fy - my) ** 2
        o_ref[...] = jnp.min(jnp.sqrt(d2 + 1e-12), axis=1, keepdims=True)

    return pl.pallas_call(
        body,
        grid=(n // rb,),
        in_specs=[
            pl.BlockSpec((rb, 2), lambda i: (i, 0)),
            pl.BlockSpec((2, nm), lambda i: (0, 0)),
        ],
        out_specs=pl.BlockSpec((rb, 1), lambda i: (i, 0)),
        out_shape=jax.ShapeDtypeStruct((n, 1), jnp.float32),
    )(fine0, mkt)


def _knn_interp(fn, nodest, cyflat, n_per, nb, rb=400):
    n = fn.shape[0]
    nc = nodest.shape[1]

    def body(f_ref, nd_ref, cy_ref, o_ref):
        i = pl.program_id(0)
        fx = f_ref[:, 0:1]
        fy = f_ref[:, 1:2]
        nx = nd_ref[0:1, :]
        ny = nd_ref[1:2, :]
        d2 = (fx - nx) ** 2 + (fy - ny) ** 2
        iot = lax.broadcasted_iota(jnp.int32, (rb, nc), 1)
        wacc = jnp.zeros((rb, nc), jnp.float32)
        wsum = jnp.zeros((rb, 1), jnp.float32)
        d2w = d2
        for _ in range(3):
            m = jnp.min(d2w, axis=1, keepdims=True)
            idx = jnp.min(jnp.where(d2w == m, iot, nc), axis=1, keepdims=True)
            oh = iot == idx
            w = 1.0 / (m + 1e-16)
            wacc = wacc + jnp.where(oh, w, 0.0)
            wsum = wsum + w
            d2w = jnp.where(oh, 1e30, d2w)
        row_batch = (i * rb + lax.broadcasted_iota(jnp.int32, (rb, 1), 0)) // n_per
        col_batch = lax.broadcasted_iota(jnp.int32, (rb, nb * nc), 1) // nc
        gate = col_batch == row_batch
        wacc4 = jnp.where(gate, jnp.tile(wacc, (1, nb)), 0.0)
        o_ref[...] = jnp.dot(wacc4, cy_ref[...],
                             preferred_element_type=jnp.float32) / wsum

    return pl.pallas_call(
        body,
        grid=(n // rb,),
        in_specs=[
            pl.BlockSpec((rb, 2), lambda i: (i, 0)),
            pl.BlockSpec((2, nc), lambda i: (0, 0)),
            pl.BlockSpec((nb * nc, 3), lambda i: (0, 0)),
        ],
        out_specs=pl.BlockSpec((rb, 3), lambda i: (i, 0)),
        out_shape=jax.ShapeDtypeStruct((n, 3), jnp.float32),
    )(fn, nodest, cyflat)


def kernel(x, edge_index, batch_ids, aoa, mach_or_reynolds, marker_inds, nodes,
           Wp0, bp0, Wp1, bp1, Wp2, bp2, We0, be0, We1, be1, We2, be2):
    batch_size = aoa.shape[0]
    n_per = x.shape[0] // batch_size

    row = edge_index[0].astype(jnp.int32)
    col = edge_index[1].astype(jnp.int32)
    pad = EP - E
    row2 = jnp.concatenate([row, jnp.zeros((pad,), jnp.int32)]).reshape(EP // CH, CH)
    col2 = jnp.concatenate([col, jnp.full((pad,), DUMMY_COL, jnp.int32)]
                           ).reshape(EP // CH, CH)
    zeros = jnp.zeros((NP, HH), jnp.float32)

    ones_tab = jnp.ones((N_FINE, HH), jnp.float32)
    deg_parts = _sc_agg(ones_tab, row2, col2, zeros)[:, :N_FINE, :]
    dinv = _dinv_from_deg(deg_parts)

    def conv(xin, w, b, relu):
        g = _mm_scale(xin, w, dinv)
        a0 = _sc_agg(g[:, :HH], row2, col2, zeros)[:, :N_FINE, :]
        a1 = _sc_agg(g[:, HH:], row2, col2, zeros)[:, :N_FINE, :]
        return _combine(a0, a1, g, dinv, b.reshape(1, H), relu)

    mkt = x[marker_inds, :2].T
    n_pad = ((n_per + 511) // 512) * 512
    fine0p = jnp.pad(x[:n_per, :2], ((0, n_pad - n_per), (0, 0)))
    sdf = _sdf(fine0p, mkt)[:n_per]
    fine_x = jnp.concatenate([x, jnp.tile(sdf, (batch_size, 1))], axis=1)

    fine_x = conv(fine_x, Wp0, bp0, True)
    fine_x = conv(fine_x, Wp1, bp1, True)
    fine_x = conv(fine_x, Wp2, bp2, True)

    cx = nodes[:, 0]
    cy = nodes[:, 1]
    a = aoa[:, None]
    m = mach_or_reynolds[:, None]
    y0 = jnp.sin(3.0 * cx)[None, :] * a + m * cy[None, :]
    y1 = jnp.cos(3.0 * cy)[None, :] * m + a * cx[None, :]
    y2 = jnp.sin(cx + cy)[None, :] * (a + m)
    cyflat = jnp.stack([y0.reshape(-1), y1.reshape(-1), y2.reshape(-1)], axis=1)

    fine_y = _knn_interp(x[:, :2], nodes.T, cyflat, n_per, batch_size)
    fine_y = jnp.concatenate([fine_y, fine_x], axis=1)

    fine_y = conv(fine_y, We0, be0, True)
    fine_y = conv(fine_y, We1, be1, True)

    out_w = We2.shape[1]
    w32 = jnp.pad(We2, ((0, 0), (0, HH - out_w)))
    b32 = jnp.pad(be2, (0, HH - out_w))
    g = _mm_scale(fine_y, w32, dinv)
    a0 = _sc_agg(g, row2, col2, zeros)[:, :N_FINE, :]
    out = _combine1(a0, g, dinv, b32.reshape(1, HH))
    return out[:, :out_w]


# speedup vs baseline: 9.3286x; 1.0031x over previous
import functools

import jax
import jax.numpy as jnp
from jax import lax
from jax.experimental import pallas as pl
from jax.experimental.pallas import tpu as pltpu, tpu_sc as plsc

N_FINE = 50000
NP = 50048
E = 800000
EP = 819200
CH = 256
CHUNKS = EP // (32 * CH)
GRP = 4
NGRP = CHUNKS // GRP
ROWS_PER_TILE = NP // 16
H = 64
HH = 32
DUMMY_COL = N_FINE + 8


def _sc_agg(table, row2, col2, zeros):
    mesh = plsc.VectorSubcoreMesh(core_axis_name="c", subcore_axis_name="s")

    @functools.partial(
        pl.kernel,
        mesh=mesh,
        out_type=jax.ShapeDtypeStruct((2, NP, HH), jnp.float32),
        scratch_types=[
            pltpu.VMEM((2, GRP, CH), jnp.int32),
            pltpu.VMEM((2, GRP, CH), jnp.int32),
            pltpu.VMEM((2, CH, HH), jnp.float32),
            pltpu.VMEM_SHARED((NP, HH), jnp.float32),
            pltpu.SemaphoreType.DMA,
            pltpu.SemaphoreType.DMA,
        ],
        compiler_params=pltpu.CompilerParams(use_tc_tiling_on_sc=False),
    )
    def k(tab_hbm, row_hbm, col_hbm, z_hbm, out_hbm,
          row_g, col_g, rows_v, acc, isem, gsem):
        c = lax.axis_index("c")
        s = lax.axis_index("s")
        wid = c * 16 + s
        pltpu.sync_copy(z_hbm.at[pl.ds(s * ROWS_PER_TILE, ROWS_PER_TILE)],
                        acc.at[pl.ds(s * ROWS_PER_TILE, ROWS_PER_TILE)])
        plsc.subcore_barrier()
        grow0 = wid * CHUNKS

        def load_idx(g, slot):
            off = grow0 + g * GRP
            pltpu.async_copy(row_hbm.at[pl.ds(off, GRP)], row_g.at[slot], isem)
            pltpu.async_copy(col_hbm.at[pl.ds(off, GRP)], col_g.at[slot], isem)

        load_idx(0, 0)

        def group(g, carry):
            slot = lax.rem(g, 2)
            pltpu.make_async_copy(row_hbm.at[pl.ds(0, GRP)], row_g.at[slot], isem).wait()
            pltpu.make_async_copy(col_hbm.at[pl.ds(0, GRP)], col_g.at[slot], isem).wait()

            @pl.when(g < NGRP - 1)
            def _():
                load_idx(g + 1, 1 - slot)

            pltpu.async_copy(tab_hbm.at[row_g.at[slot, 0]], rows_v.at[0], gsem)
            for b in range(GRP):
                if b + 1 < GRP:
                    pltpu.async_copy(tab_hbm.at[row_g.at[slot, b + 1]],
                                     rows_v.at[(b + 1) % 2], gsem)
                pltpu.make_async_copy(tab_hbm.at[row_g.at[slot, b]],
                                      rows_v.at[b % 2], gsem).wait()
                pltpu.sync_copy(rows_v.at[b % 2], acc.at[col_g.at[slot, b]], add=True)
            return carry

        lax.fori_loop(0, NGRP, group, 0)
        plsc.subcore_barrier()
        pltpu.sync_copy(acc.at[pl.ds(s * ROWS_PER_TILE, ROWS_PER_TILE)],
                        out_hbm.at[c, pl.ds(s * ROWS_PER_TILE, ROWS_PER_TILE)])

    return k(table, row2, col2, zeros)


def _mm_scale(x, w, dinv, rb=1000):
    n, kd = x.shape
    ho = w.shape[1]

    def body(x_ref, w_ref, d_ref, o_ref):
        o_ref[...] = d_ref[...] * jnp.dot(
            x_ref[...], w_ref[...], preferred_element_type=jnp.float32)

    return pl.pallas_call(
        body,
        grid=(n // rb,),
        in_specs=[
            pl.BlockSpec((rb, kd), lambda i: (i, 0)),
            pl.BlockSpec((kd, ho), lambda i: (0, 0)),
            pl.BlockSpec((rb, 1), lambda i: (i, 0)),
        ],
        out_specs=pl.BlockSpec((rb, ho), lambda i: (i, 0)),
        out_shape=jax.ShapeDtypeStruct((n, ho), jnp.float32),
    )(x, w, dinv)


def _combine(a0, a1, g, dinv, b, relu, rb=1000):
    n = g.shape[0]

    def body(a0_ref, a1_ref, g_ref, d_ref, b_ref, o_ref):
        lo = a0_ref[0] + a0_ref[1]
        hi = a1_ref[0] + a1_ref[1]
        agg = jnp.concatenate([lo, hi], axis=1)
        y = d_ref[...] * (agg + g_ref[...]) + b_ref[...]
        if relu:
            y = jnp.maximum(y, 0.0)
        o_ref[...] = y

    return pl.pallas_call(
        body,
        grid=(n // rb,),
        in_specs=[
            pl.BlockSpec((2, rb, HH), lambda i: (0, i, 0)),
            pl.BlockSpec((2, rb, HH), lambda i: (0, i, 0)),
            pl.BlockSpec((rb, H), lambda i: (i, 0)),
            pl.BlockSpec((rb, 1), lambda i: (i, 0)),
            pl.BlockSpec((1, H), lambda i: (0, 0)),
        ],
        out_specs=pl.BlockSpec((rb, H), lambda i: (i, 0)),
        out_shape=jax.ShapeDtypeStruct((n, H), jnp.float32),
    )(a0, a1, g, dinv, b)


def _combine1(a0, g, dinv, b, rb=1000):
    n = g.shape[0]

    def body(a0_ref, g_ref, d_ref, b_ref, o_ref):
        agg = a0_ref[0] + a0_ref[1]
        o_ref[...] = d_ref[...] * (agg + g_ref[...]) + b_ref[...]

    return pl.pallas_call(
        body,
        grid=(n // rb,),
        in_specs=[
            pl.BlockSpec((2, rb, HH), lambda i: (0, i, 0)),
            pl.BlockSpec((rb, HH), lambda i: (i, 0)),
            pl.BlockSpec((rb, 1), lambda i: (i, 0)),
            pl.BlockSpec((1, HH), lambda i: (0, 0)),
        ],
        out_specs=pl.BlockSpec((rb, HH), lambda i: (i, 0)),
        out_shape=jax.ShapeDtypeStruct((n, HH), jnp.float32),
    )(a0, g, dinv, b)


def _dinv_from_deg(d, rb=1000):
    n = N_FINE

    def body(d_ref, o_ref):
        deg = d_ref[0, :, 0:1] + d_ref[1, :, 0:1] + 1.0
        o_ref[...] = lax.rsqrt(deg)

    return pl.pallas_call(
        body,
        grid=(n // rb,),
        in_specs=[pl.BlockSpec((2, rb, HH), lambda i: (0, i, 0))],
        out_specs=pl.BlockSpec((rb, 1), lambda i: (i, 0)),
        out_shape=jax.ShapeDtypeStruct((n, 1), jnp.float32),
    )(d)


def _sdf(fine0, mkt, rb=512):
    n = fine0.shape[0]
    nm = mkt.shape[1]

    def body(f_ref, m_ref, o_ref):
        fx = f_ref[:, 0:1]
        fy = f_ref[:, 1:2]
        mx = m_ref[0:1, :]
        my = m_ref[1:2, :]
        d2 = (fx - mx) ** 2 + (fy - my) ** 2
        o_ref[...] = jnp.min(jnp.sqrt(d2 + 1e-12), axis=1, keepdims=True)

    return pl.pallas_call(
        body,
        grid=(n // rb,),
        in_specs=[
            pl.BlockSpec((rb, 2), lambda i: (i, 0)),
            pl.BlockSpec((2, nm), lambda i: (0, 0)),
        ],
        out_specs=pl.BlockSpec((rb, 1), lambda i: (i, 0)),
        out_shape=jax.ShapeDtypeStruct((n, 1), jnp.float32),
    )(fine0, mkt)


def _knn_interp(fn, nodest, cyflat, n_per, nb, rb=400):
    n = fn.shape[0]
    nc = nodest.shape[1]

    def body(f_ref, nd_ref, cy_ref, o_ref):
        i = pl.program_id(0)
        fx = f_ref[:, 0:1]
        fy = f_ref[:, 1:2]
        nx = nd_ref[0:1, :]
        ny = nd_ref[1:2, :]
        d2 = (fx - nx) ** 2 + (fy - ny) ** 2
        iot = lax.broadcasted_iota(jnp.int32, (rb, nc), 1)
        wacc = jnp.zeros((rb, nc), jnp.float32)
        wsum = jnp.zeros((rb, 1), jnp.float32)
        d2w = d2
        for _ in range(3):
            m = jnp.min(d2w, axis=1, keepdims=True)
            idx = jnp.min(jnp.where(d2w == m, iot, nc), axis=1, keepdims=True)
            oh = iot == idx
            w = 1.0 / (m + 1e-16)
            wacc = wacc + jnp.where(oh, w, 0.0)
            wsum = wsum + w
            d2w = jnp.where(oh, 1e30, d2w)
        row_batch = (i * rb + lax.broadcasted_iota(jnp.int32, (rb, 1), 0)) // n_per
        col_batch = lax.broadcasted_iota(jnp.int32, (rb, nb * nc), 1) // nc
        gate = col_batch == row_batch
        wacc4 = jnp.where(gate, jnp.tile(wacc, (1, nb)), 0.0)
        o_ref[...] = jnp.dot(wacc4, cy_ref[...],
                             preferred_element_type=jnp.float32) / wsum

    return pl.pallas_call(
        body,
        grid=(n // rb,),
        in_specs=[
            pl.BlockSpec((rb, 2), lambda i: (i, 0)),
            pl.BlockSpec((2, nc), lambda i: (0, 0)),
            pl.BlockSpec((nb * nc, 3), lambda i: (0, 0)),
        ],
        out_specs=pl.BlockSpec((rb, 3), lambda i: (i, 0)),
        out_shape=jax.ShapeDtypeStruct((n, 3), jnp.float32),
    )(fn, nodest, cyflat)


def kernel(x, edge_index, batch_ids, aoa, mach_or_reynolds, marker_inds, nodes,
           Wp0, bp0, Wp1, bp1, Wp2, bp2, We0, be0, We1, be1, We2, be2):
    batch_size = aoa.shape[0]
    n_per = x.shape[0] // batch_size

    row = edge_index[0].astype(jnp.int32)
    col = edge_index[1].astype(jnp.int32)
    pad = EP - E
    row2 = jnp.concatenate([row, jnp.zeros((pad,), jnp.int32)]).reshape(EP // CH, CH)
    col2 = jnp.concatenate([col, jnp.full((pad,), DUMMY_COL, jnp.int32)]
                           ).reshape(EP // CH, CH)
    zeros = jnp.zeros((NP, HH), jnp.float32)

    ones_tab = jnp.ones((N_FINE, HH), jnp.float32)
    deg_parts = _sc_agg(ones_tab, row2, col2, zeros)[:, :N_FINE, :]
    dinv = _dinv_from_deg(deg_parts)

    def conv(xin, w, b, relu):
        g = _mm_scale(xin, w, dinv)
        a0 = _sc_agg(g[:, :HH], row2, col2, zeros)[:, :N_FINE, :]
        a1 = _sc_agg(g[:, HH:], row2, col2, zeros)[:, :N_FINE, :]
        return _combine(a0, a1, g, dinv, b.reshape(1, H), relu)

    mkt = x[marker_inds, :2].T
    n_pad = ((n_per + 511) // 512) * 512
    fine0p = jnp.pad(x[:n_per, :2], ((0, n_pad - n_per), (0, 0)))
    sdf = _sdf(fine0p, mkt)[:n_per]
    fine_x = jnp.concatenate([x, jnp.tile(sdf, (batch_size, 1))], axis=1)

    fine_x = conv(fine_x, Wp0, bp0, True)
    fine_x = conv(fine_x, Wp1, bp1, True)
    fine_x = conv(fine_x, Wp2, bp2, True)

    cx = nodes[:, 0]
    cy = nodes[:, 1]
    a = aoa[:, None]
    m = mach_or_reynolds[:, None]
    y0 = jnp.sin(3.0 * cx)[None, :] * a + m * cy[None, :]
    y1 = jnp.cos(3.0 * cy)[None, :] * m + a * cx[None, :]
    y2 = jnp.sin(cx + cy)[None, :] * (a + m)
    cyflat = jnp.stack([y0.reshape(-1), y1.reshape(-1), y2.reshape(-1)], axis=1)

    fine_y = _knn_interp(x[:, :2], nodes.T, cyflat, n_per, batch_size)
    fine_y = jnp.concatenate([fine_y, fine_x], axis=1)

    fine_y = conv(fine_y, We0, be0, True)
    fine_y = conv(fine_y, We1, be1, True)

    out_w = We2.shape[1]
    w32 = jnp.pad(We2, ((0, 0), (0, HH - out_w)))
    b32 = jnp.pad(be2, (0, HH - out_w))
    g = _mm_scale(fine_y, w32, dinv)
    a0 = _sc_agg(g, row2, col2, zeros)[:, :N_FINE, :]
    out = _combine1(a0, g, dinv, b32.reshape(1, HH))
    return out[:, :out_w]


# speedup vs baseline: 10.0809x; 1.0806x over previous
import functools

import jax
import jax.numpy as jnp
from jax import lax
from jax.experimental import pallas as pl
from jax.experimental.pallas import tpu as pltpu, tpu_sc as plsc

N_FINE = 50000
NP = 50048
E = 800000
EP = 819200
CH = 256
CHUNKS = EP // (32 * CH)
GRP = 4
NGRP = CHUNKS // GRP
ROWS_PER_TILE = NP // 16
H = 64
HH = 32
DUMMY_COL = N_FINE + 8


def _sc_agg(table, row2, col2, zeros, hh=HH):
    mesh = plsc.VectorSubcoreMesh(core_axis_name="c", subcore_axis_name="s")

    @functools.partial(
        pl.kernel,
        mesh=mesh,
        out_type=jax.ShapeDtypeStruct((2, NP, hh), jnp.float32),
        scratch_types=[
            pltpu.VMEM((2, GRP, CH), jnp.int32),
            pltpu.VMEM((2, GRP, CH), jnp.int32),
            pltpu.VMEM((2, CH, hh), jnp.float32),
            pltpu.VMEM_SHARED((NP, hh), jnp.float32),
            pltpu.SemaphoreType.DMA,
            pltpu.SemaphoreType.DMA,
        ],
        compiler_params=pltpu.CompilerParams(use_tc_tiling_on_sc=False),
    )
    def k(tab_hbm, row_hbm, col_hbm, z_hbm, out_hbm,
          row_g, col_g, rows_v, acc, isem, gsem):
        c = lax.axis_index("c")
        s = lax.axis_index("s")
        wid = c * 16 + s
        pltpu.sync_copy(z_hbm.at[pl.ds(s * ROWS_PER_TILE, ROWS_PER_TILE)],
                        acc.at[pl.ds(s * ROWS_PER_TILE, ROWS_PER_TILE)])
        plsc.subcore_barrier()
        grow0 = wid * CHUNKS

        def load_idx(g, slot):
            off = grow0 + g * GRP
            pltpu.async_copy(row_hbm.at[pl.ds(off, GRP)], row_g.at[slot], isem)
            pltpu.async_copy(col_hbm.at[pl.ds(off, GRP)], col_g.at[slot], isem)

        load_idx(0, 0)

        def group(g, carry):
            slot = lax.rem(g, 2)
            pltpu.make_async_copy(row_hbm.at[pl.ds(0, GRP)], row_g.at[slot], isem).wait()
            pltpu.make_async_copy(col_hbm.at[pl.ds(0, GRP)], col_g.at[slot], isem).wait()

            @pl.when(g < NGRP - 1)
            def _():
                load_idx(g + 1, 1 - slot)

            pltpu.async_copy(tab_hbm.at[row_g.at[slot, 0]], rows_v.at[0], gsem)
            for b in range(GRP):
                if b + 1 < GRP:
                    pltpu.async_copy(tab_hbm.at[row_g.at[slot, b + 1]],
                                     rows_v.at[(b + 1) % 2], gsem)
                pltpu.make_async_copy(tab_hbm.at[row_g.at[slot, b]],
                                      rows_v.at[b % 2], gsem).wait()
                pltpu.sync_copy(rows_v.at[b % 2], acc.at[col_g.at[slot, b]], add=True)
            return carry

        lax.fori_loop(0, NGRP, group, 0)
        plsc.subcore_barrier()
        pltpu.sync_copy(acc.at[pl.ds(s * ROWS_PER_TILE, ROWS_PER_TILE)],
                        out_hbm.at[c, pl.ds(s * ROWS_PER_TILE, ROWS_PER_TILE)])

    return k(table, row2, col2, zeros)


def _mm_scale(x, w, dinv, rb=1000):
    n, kd = x.shape
    ho = w.shape[1]

    def body(x_ref, w_ref, d_ref, o_ref):
        o_ref[...] = d_ref[...] * jnp.dot(
            x_ref[...], w_ref[...], preferred_element_type=jnp.float32)

    return pl.pallas_call(
        body,
        grid=(n // rb,),
        in_specs=[
            pl.BlockSpec((rb, kd), lambda i: (i, 0)),
            pl.BlockSpec((kd, ho), lambda i: (0, 0)),
            pl.BlockSpec((rb, 1), lambda i: (i, 0)),
        ],
        out_specs=pl.BlockSpec((rb, ho), lambda i: (i, 0)),
        out_shape=jax.ShapeDtypeStruct((n, ho), jnp.float32),
    )(x, w, dinv)


def _combine(a0, a1, g, dinv, b, relu, rb=1000):
    n = g.shape[0]

    def body(a0_ref, a1_ref, g_ref, d_ref, b_ref, o_ref):
        lo = a0_ref[0] + a0_ref[1]
        hi = a1_ref[0] + a1_ref[1]
        agg = jnp.concatenate([lo, hi], axis=1)
        y = d_ref[...] * (agg + g_ref[...]) + b_ref[...]
        if relu:
            y = jnp.maximum(y, 0.0)
        o_ref[...] = y

    return pl.pallas_call(
        body,
        grid=(n // rb,),
        in_specs=[
            pl.BlockSpec((2, rb, HH), lambda i: (0, i, 0)),
            pl.BlockSpec((2, rb, HH), lambda i: (0, i, 0)),
            pl.BlockSpec((rb, H), lambda i: (i, 0)),
            pl.BlockSpec((rb, 1), lambda i: (i, 0)),
            pl.BlockSpec((1, H), lambda i: (0, 0)),
        ],
        out_specs=pl.BlockSpec((rb, H), lambda i: (i, 0)),
        out_shape=jax.ShapeDtypeStruct((n, H), jnp.float32),
    )(a0, a1, g, dinv, b)


def _combine1(a0, g, dinv, b, rb=1000):
    n = g.shape[0]

    def body(a0_ref, g_ref, d_ref, b_ref, o_ref):
        agg = a0_ref[0] + a0_ref[1]
        o_ref[...] = d_ref[...] * (agg + g_ref[...]) + b_ref[...]

    return pl.pallas_call(
        body,
        grid=(n // rb,),
        in_specs=[
            pl.BlockSpec((2, rb, HH), lambda i: (0, i, 0)),
            pl.BlockSpec((rb, HH), lambda i: (i, 0)),
            pl.BlockSpec((rb, 1), lambda i: (i, 0)),
            pl.BlockSpec((1, HH), lambda i: (0, 0)),
        ],
        out_specs=pl.BlockSpec((rb, HH), lambda i: (i, 0)),
        out_shape=jax.ShapeDtypeStruct((n, HH), jnp.float32),
    )(a0, g, dinv, b)


def _dinv_from_deg(d, rb=1000):
    n = N_FINE

    def body(d_ref, o_ref):
        deg = d_ref[0, :, 0:1] + d_ref[1, :, 0:1] + 1.0
        o_ref[...] = lax.rsqrt(deg)

    return pl.pallas_call(
        body,
        grid=(n // rb,),
        in_specs=[pl.BlockSpec((2, rb, 16), lambda i: (0, i, 0))],
        out_specs=pl.BlockSpec((rb, 1), lambda i: (i, 0)),
        out_shape=jax.ShapeDtypeStruct((n, 1), jnp.float32),
    )(d)


def _sdf(fine0, mkt, rb=512):
    n = fine0.shape[0]
    nm = mkt.shape[1]

    def body(f_ref, m_ref, o_ref):
        fx = f_ref[:, 0:1]
        fy = f_ref[:, 1:2]
        mx = m_ref[0:1, :]
        my = m_ref[1:2, :]
        d2 = (fx - mx) ** 2 + (fy - my) ** 2
        o_ref[...] = jnp.min(jnp.sqrt(d2 + 1e-12), axis=1, keepdims=True)

    return pl.pallas_call(
        body,
        grid=(n // rb,),
        in_specs=[
            pl.BlockSpec((rb, 2), lambda i: (i, 0)),
            pl.BlockSpec((2, nm), lambda i: (0, 0)),
        ],
        out_specs=pl.BlockSpec((rb, 1), lambda i: (i, 0)),
        out_shape=jax.ShapeDtypeStruct((n, 1), jnp.float32),
    )(fine0, mkt)


def _knn_interp(fn, nodest, cyflat, n_per, nb, rb=400):
    n = fn.shape[0]
    nc = nodest.shape[1]

    def body(f_ref, nd_ref, cy_ref, o_ref):
        i = pl.program_id(0)
        fx = f_ref[:, 0:1]
        fy = f_ref[:, 1:2]
        nx = nd_ref[0:1, :]
        ny = nd_ref[1:2, :]
        d2 = (fx - nx) ** 2 + (fy - ny) ** 2
        iot = lax.broadcasted_iota(jnp.int32, (rb, nc), 1)
        wacc = jnp.zeros((rb, nc), jnp.float32)
        wsum = jnp.zeros((rb, 1), jnp.float32)
        d2w = d2
        for _ in range(3):
            m = jnp.min(d2w, axis=1, keepdims=True)
            idx = jnp.min(jnp.where(d2w == m, iot, nc), axis=1, keepdims=True)
            oh = iot == idx
            w = 1.0 / (m + 1e-16)
            wacc = wacc + jnp.where(oh, w, 0.0)
            wsum = wsum + w
            d2w = jnp.where(oh, 1e30, d2w)
        row_batch = (i * rb + lax.broadcasted_iota(jnp.int32, (rb, 1), 0)) // n_per
        col_batch = lax.broadcasted_iota(jnp.int32, (rb, nb * nc), 1) // nc
        gate = col_batch == row_batch
        wacc4 = jnp.where(gate, jnp.tile(wacc, (1, nb)), 0.0)
        o_ref[...] = jnp.dot(wacc4, cy_ref[...],
                             preferred_element_type=jnp.float32) / wsum

    return pl.pallas_call(
        body,
        grid=(n // rb,),
        in_specs=[
            pl.BlockSpec((rb, 2), lambda i: (i, 0)),
            pl.BlockSpec((2, nc), lambda i: (0, 0)),
            pl.BlockSpec((nb * nc, 3), lambda i: (0, 0)),
        ],
        out_specs=pl.BlockSpec((rb, 3), lambda i: (i, 0)),
        out_shape=jax.ShapeDtypeStruct((n, 3), jnp.float32),
    )(fn, nodest, cyflat)


def kernel(x, edge_index, batch_ids, aoa, mach_or_reynolds, marker_inds, nodes,
           Wp0, bp0, Wp1, bp1, Wp2, bp2, We0, be0, We1, be1, We2, be2):
    batch_size = aoa.shape[0]
    n_per = x.shape[0] // batch_size

    row = edge_index[0].astype(jnp.int32)
    col = edge_index[1].astype(jnp.int32)
    pad = EP - E
    row2 = jnp.concatenate([row, jnp.zeros((pad,), jnp.int32)]).reshape(EP // CH, CH)
    col2 = jnp.concatenate([col, jnp.full((pad,), DUMMY_COL, jnp.int32)]
                           ).reshape(EP // CH, CH)
    zeros = jnp.zeros((NP, HH), jnp.float32)

    ones_tab = jnp.ones((N_FINE, 16), jnp.float32)
    zeros16 = jnp.zeros((NP, 16), jnp.float32)
    deg_parts = _sc_agg(ones_tab, row2, col2, zeros16, hh=16)[:, :N_FINE, :]
    dinv = _dinv_from_deg(deg_parts)

    def conv(xin, w, b, relu):
        g = _mm_scale(xin, w, dinv)
        a0 = _sc_agg(g[:, :HH], row2, col2, zeros)[:, :N_FINE, :]
        a1 = _sc_agg(g[:, HH:], row2, col2, zeros)[:, :N_FINE, :]
        return _combine(a0, a1, g, dinv, b.reshape(1, H), relu)

    mkt = x[marker_inds, :2].T
    n_pad = ((n_per + 511) // 512) * 512
    fine0p = jnp.pad(x[:n_per, :2], ((0, n_pad - n_per), (0, 0)))
    sdf = _sdf(fine0p, mkt)[:n_per]
    fine_x = jnp.concatenate([x, jnp.tile(sdf, (batch_size, 1))], axis=1)

    fine_x = conv(fine_x, Wp0, bp0, True)
    fine_x = conv(fine_x, Wp1, bp1, True)
    fine_x = conv(fine_x, Wp2, bp2, True)

    cx = nodes[:, 0]
    cy = nodes[:, 1]
    a = aoa[:, None]
    m = mach_or_reynolds[:, None]
    y0 = jnp.sin(3.0 * cx)[None, :] * a + m * cy[None, :]
    y1 = jnp.cos(3.0 * cy)[None, :] * m + a * cx[None, :]
    y2 = jnp.sin(cx + cy)[None, :] * (a + m)
    cyflat = jnp.stack([y0.reshape(-1), y1.reshape(-1), y2.reshape(-1)], axis=1)

    fine_y = _knn_interp(x[:, :2], nodes.T, cyflat, n_per, batch_size)
    fine_y = jnp.concatenate([fine_y, fine_x], axis=1)

    fine_y = conv(fine_y, We0, be0, True)
    fine_y = conv(fine_y, We1, be1, True)

    out_w = We2.shape[1]
    w32 = jnp.pad(We2, ((0, 0), (0, HH - out_w)))
    b32 = jnp.pad(be2, (0, HH - out_w))
    g = _mm_scale(fine_y, w32, dinv)
    a0 = _sc_agg(g, row2, col2, zeros)[:, :N_FINE, :]
    out = _combine1(a0, g, dinv, b32.reshape(1, HH))
    return out[:, :out_w]
